# baseline (speedup 1.0000x reference)
import jax
import jax.numpy as jnp
from jax import lax
from jax.experimental import pallas as pl
from jax.experimental.pallas import tpu as pltpu
from jax.experimental.pallas import tpu_sc as plsc

N = 196
C = 384
C2 = 192
K = 8
NPI = 256
NJR = 200
NP2 = 256

NC, NS = 2, 16
NW = NC * NS


def _dist_body(xf, wm, ws, m, difft_out):
    a = jax.lax.dot_general(
        m[...], m[...], (((0,), (0,)), ((), ())),
        preferred_element_type=jnp.float32)
    ev = jnp.exp(jax.lax.dot_general(
        ws[...], xf[0], (((1,), (0,)), ((), ())),
        preferred_element_type=jnp.float32))
    ft = jnp.exp(-jax.lax.dot_general(
        wm[...], xf[0], (((1,), (0,)), ((), ())),
        preferred_element_type=jnp.float32))

    s_cat = jnp.concatenate(
        [1.0 / (1.0 + ev[:, j:j + 1] * ft) for j in range(NJR)],
        axis=1)
    t_cat = jax.lax.dot_general(
        a, s_cat, (((1,), (0,)), ((), ())),
        preferred_element_type=jnp.float32)
    rows = [jnp.sum(t_cat[:, j * NPI:(j + 1) * NPI] *
                    s_cat[:, j * NPI:(j + 1) * NPI], axis=0, keepdims=True)
            for j in range(NJR)]
    d2b = jnp.maximum(jnp.concatenate(rows, axis=0), 0.0)
    difft_out[0, pl.ds(0, NJR), :] = jax.nn.sigmoid(-jnp.sqrt(d2b))


def _sc_topk(difft_hbm, val_hbm, idx_hbm, in_v, val_v, idx_v, sem):
    wid = lax.axis_index("s") * NC + lax.axis_index("c")
    b = wid // NS
    g = wid % NS
    base = b * (NP2 * NPI) + g * 16
    descs = [
        pltpu.async_copy(difft_hbm.at[pl.ds(base + j * NPI, 16)],
                         in_v.at[pl.ds(j * 16, 16)], sem)
        for j in range(N)
    ]
    for d in descs:
        d.wait()
    minf = jnp.full((16,), -jnp.inf, dtype=jnp.float32)
    zero = jnp.zeros((16,), dtype=jnp.int32)

    def step(j, carry):
        bv = list(carry[:K])
        bi = list(carry[K:])
        c = in_v[pl.ds(j * 16, 16)]
        ci = zero + j
        for k in range(K):
            cond = c > bv[k]
            nv = jnp.where(cond, c, bv[k])
            c = jnp.where(cond, bv[k], c)
            ni = jnp.where(cond, ci, bi[k])
            ci = jnp.where(cond, bi[k], ci)
            bv[k], bi[k] = nv, ni
        return tuple(bv) + tuple(bi)

    init = tuple(minf for _ in range(K)) + tuple(zero for _ in range(K))
    res = lax.fori_loop(0, N, step, init)
    for k in range(K):
        val_v[pl.ds(k * 16, 16)] = -res[k]
        idx_v[pl.ds(k * 16, 16)] = res[K + k]
    pltpu.sync_copy(val_v, val_hbm.at[pl.ds(wid * (K * 16), K * 16)])
    pltpu.sync_copy(idx_v, idx_hbm.at[pl.ds(wid * (K * 16), K * 16)])


@jax.jit
def kernel(X, Wm, Ws, M):
    B = X.shape[0]
    Xf = X.reshape(B, C, N)
    Xfp = jnp.pad(Xf, ((0, 0), (0, 0), (0, NPI - N)))

    difft = pl.pallas_call(
        _dist_body,
        grid=(B,),
        in_specs=[
            pl.BlockSpec((1, C, NPI), lambda b: (b, 0, 0)),
            pl.BlockSpec((C2, C), lambda b: (0, 0)),
            pl.BlockSpec((C2, C), lambda b: (0, 0)),
            pl.BlockSpec((C2, C2), lambda b: (0, 0)),
        ],
        out_specs=pl.BlockSpec((1, NP2, NPI), lambda b: (b, 0, 0)),
        out_shape=jax.ShapeDtypeStruct((B, NP2, NPI), jnp.float32),
    )(Xfp, Wm, Ws, M)

    topk = pl.kernel(
        _sc_topk,
        out_type=[
            jax.ShapeDtypeStruct((NW * K * 16,), jnp.float32),
            jax.ShapeDtypeStruct((NW * K * 16,), jnp.int32),
        ],
        mesh=plsc.VectorSubcoreMesh(core_axis_name="c", subcore_axis_name="s"),
        scratch_types=[
            pltpu.VMEM((N * 16,), jnp.float32),
            pltpu.VMEM((K * 16,), jnp.float32),
            pltpu.VMEM((K * 16,), jnp.int32),
            pltpu.SemaphoreType.DMA,
        ],
    )
    val_f, idx_f = topk(difft.reshape(B * NP2 * NPI))

    val4 = val_f.reshape(B, NS, K, 16).transpose(0, 1, 3, 2)
    idx4 = idx_f.reshape(B, NS, K, 16).transpose(0, 1, 3, 2)
    index = idx4.reshape(B, NPI, K)[:, :N, :].reshape(B, N * K)
    value = val4.reshape(B, NPI, K)[:, :N, :].reshape(B, N * K, 1)
    return (index, value)


# speedup vs baseline: 1.7043x; 1.7043x over previous
import jax
import jax.numpy as jnp
from jax.experimental import pallas as pl
from jax.experimental.pallas import tpu as pltpu

N = 196
C = 384
C2 = 192
K = 8
NPJ = 256
NP = 224
NROW = 200
R = 2 * NP


def _srknn_body(xf, wm, ws, m, idx_out, val_out, diff_s):
    b = pl.program_id(0)
    nb = pl.num_programs(0)

    @pl.when(b == 0)
    def _():
        diff_s[...] = jnp.full((R, NPJ), -jnp.inf, dtype=jnp.float32)

    a = jax.lax.dot_general(
        m[...], m[...], (((0,), (0,)), ((), ())),
        preferred_element_type=jnp.float32)
    ev = jnp.exp(jax.lax.dot_general(
        ws[...], xf[0], (((1,), (0,)), ((), ())),
        preferred_element_type=jnp.float32))
    ft = jnp.exp(-jax.lax.dot_general(
        wm[...], xf[0], (((1,), (0,)), ((), ())),
        preferred_element_type=jnp.float32))

    s_cat = jnp.concatenate(
        [1.0 / (1.0 + ft[:, r:r + 1] * ev) for r in range(NROW)],
        axis=1)
    t_cat = jax.lax.dot_general(
        a, s_cat, (((1,), (0,)), ((), ())),
        preferred_element_type=jnp.float32)
    rows = [jnp.sum(t_cat[:, r * NPJ:(r + 1) * NPJ] *
                    s_cat[:, r * NPJ:(r + 1) * NPJ], axis=0, keepdims=True)
            for r in range(NROW)]
    d2b = jnp.maximum(jnp.concatenate(rows, axis=0), 0.0)
    diff = jax.nn.sigmoid(-jnp.sqrt(d2b))

    lane_b = jax.lax.broadcasted_iota(jnp.int32, (NROW, NPJ), 1)
    diff_s[pl.ds(b * NP, NROW), :] = jnp.where(lane_b < N, diff, -jnp.inf)

    @pl.when(b == nb - 1)
    def _():
        diffall = diff_s[...]
        lane = jax.lax.broadcasted_iota(jnp.int32, (R, NPJ), 1)
        vals, idxs = [], []
        for _ in range(K):
            mx = jnp.max(diffall, axis=1, keepdims=True)
            amx = jnp.min(jnp.where(diffall == mx, lane, NPJ), axis=1,
                          keepdims=True)
            vals.append(-mx)
            idxs.append(amx)
            diffall = jnp.where(lane == amx, -jnp.inf, diffall)
        val_out[...] = jnp.concatenate(vals, axis=1)
        idx_out[...] = jnp.concatenate(idxs, axis=1)


@jax.jit
def kernel(X, Wm, Ws, M):
    B = X.shape[0]
    Xf = X.reshape(B, C, N)
    Xfp = jnp.pad(Xf, ((0, 0), (0, 0), (0, NPJ - N)))

    idx_pad, val_pad = pl.pallas_call(
        _srknn_body,
        grid=(B,),
        in_specs=[
            pl.BlockSpec((1, C, NPJ), lambda b: (b, 0, 0)),
            pl.BlockSpec((C2, C), lambda b: (0, 0)),
            pl.BlockSpec((C2, C), lambda b: (0, 0)),
            pl.BlockSpec((C2, C2), lambda b: (0, 0)),
        ],
        out_specs=[
            pl.BlockSpec((R, K), lambda b: (0, 0)),
            pl.BlockSpec((R, K), lambda b: (0, 0)),
        ],
        out_shape=[
            jax.ShapeDtypeStruct((R, K), jnp.int32),
            jax.ShapeDtypeStruct((R, K), jnp.float32),
        ],
        scratch_shapes=[
            pltpu.VMEM((R, NPJ), jnp.float32),
        ],
    )(Xfp, Wm, Ws, M)

    index = idx_pad.reshape(B, NP, K)[:, :N, :].reshape(B, N * K)
    value = val_pad.reshape(B, NP, K)[:, :N, :].reshape(B, N * K, 1)
    return (index, value)
